# baseline (speedup 1.0000x reference)
import functools

import jax
import jax.numpy as jnp
from jax import lax
from jax.experimental import pallas as pl
from jax.experimental.pallas import tpu as pltpu
from jax.experimental.pallas import tpu_sc as plsc

NC = 2
NS = 16
NW = NC * NS

N_ROWS, N_COLS = 16384, 26
B = N_ROWS * N_COLS
D = 128
BPW = B // NW
CHUNK = 128
NCHUNK = BPW // CHUNK
NBLK = B // CHUNK
NSUP = NCHUNK // 2
NBUF = 3


def _gather_body(idx_hbm, table_hbm, out_hbm, idx_v, rows_v, gsem, ssem):
    cid = lax.axis_index("c")
    sid = lax.axis_index("s")
    wid = sid * NC + cid
    pltpu.sync_copy(idx_hbm.at[wid], idx_v)
    blk0 = wid * NCHUNK

    def fire_gathers(sup, slot):
        for h in range(2):
            pltpu.async_copy(table_hbm.at[idx_v.at[2 * sup + h]],
                             rows_v.at[slot, h], gsem.at[slot])

    def wait_gathers(sup, slot):
        for h in range(2):
            pltpu.make_async_copy(
                table_hbm.at[idx_v.at[2 * sup + h]],
                rows_v.at[slot, h], gsem.at[slot]).wait()

    def fire_store(sup, slot):
        return pltpu.async_copy(
            rows_v.at[slot], out_hbm.at[pl.ds(blk0 + 2 * sup, 2)],
            ssem.at[slot])

    def wait_store(sup, slot):
        pltpu.make_async_copy(
            rows_v.at[slot], out_hbm.at[pl.ds(blk0 + 2 * sup, 2)],
            ssem.at[slot]).wait()

    fire_gathers(0, 0)
    fire_gathers(1, 1)
    wait_gathers(0, 0)
    fire_store(0, 0)
    fire_gathers(2, 2)
    wait_gathers(1, 1)
    fire_store(1, 1)
    wait_store(0, 0)
    fire_gathers(3, 0)

    def group(g, carry):
        for k in range(NBUF):
            s_ = 2 + g * NBUF + k
            b = (2 + k) % NBUF
            s2 = (b + 2) % NBUF
            wait_gathers(s_, b)
            fire_store(s_, b)
            wait_store(s_ - 1, s2)
            fire_gathers(s_ + 2, s2)
        return carry

    G = (NSUP - 2 - 2) // NBUF
    lax.fori_loop(0, G, group, 0)

    for s_ in range(2 + G * NBUF, NSUP - 2):
        b = s_ % NBUF
        s2 = (b + 2) % NBUF
        wait_gathers(s_, b)
        fire_store(s_, b)
        wait_store(s_ - 1, s2)
        fire_gathers(s_ + 2, s2)
    for s_ in range(NSUP - 2, NSUP):
        b = s_ % NBUF
        wait_gathers(s_, b)
        fire_store(s_, b)
        wait_store(s_ - 1, (s_ - 1) % NBUF)
    wait_store(NSUP - 1, (NSUP - 1) % NBUF)


@functools.partial(jax.jit, static_argnames=())
def _sc_gather(idx, table):
    kern = pl.kernel(
        _gather_body,
        out_type=jax.ShapeDtypeStruct((NBLK, CHUNK, D), jnp.float32),
        mesh=plsc.VectorSubcoreMesh(
            core_axis_name="c", subcore_axis_name="s",
            num_cores=NC, num_subcores=NS),
        scratch_types=[
            pltpu.VMEM((NCHUNK, CHUNK), jnp.int32),
            pltpu.VMEM((NBUF, 2, CHUNK, D), jnp.float32),
            pltpu.SemaphoreType.DMA((NBUF,)),
            pltpu.SemaphoreType.DMA((NBUF,)),
        ],
    )
    return kern(idx, table)


def kernel(distance, FEATURE):
    idx = jnp.transpose(distance).reshape(NW, NCHUNK, CHUNK).astype(jnp.int32)
    out = _sc_gather(idx, FEATURE)
    return out.reshape(N_COLS, N_ROWS, D).transpose(1, 0, 2)


# speedup vs baseline: 1.0157x; 1.0157x over previous
import functools

import jax
import jax.numpy as jnp
from jax import lax
from jax.experimental import pallas as pl
from jax.experimental.pallas import tpu as pltpu
from jax.experimental.pallas import tpu_sc as plsc

NC = 2
NS = 16
NW = NC * NS

N_ROWS, N_COLS = 16384, 26
B = N_ROWS * N_COLS
D = 128
BPW = B // NW
CHUNK = 128
NCHUNK = BPW // CHUNK
NBLK = B // CHUNK
NBUF = 6
LA = NBUF - 2


def _gather_body(idx_hbm, table_hbm, out_hbm, idx_v, rows_v, gsem, ssem):
    cid = lax.axis_index("c")
    sid = lax.axis_index("s")
    wid = sid * NC + cid
    pltpu.sync_copy(idx_hbm.at[wid], idx_v)
    blk0 = wid * NCHUNK

    def fire_gather(chunk, slot):
        return pltpu.async_copy(
            table_hbm.at[idx_v.at[chunk]], rows_v.at[slot, 0], gsem.at[slot])

    def fire_store(chunk, slot):
        return pltpu.async_copy(
            rows_v.at[slot], out_hbm.at[pl.ds(blk0 + chunk, 1)],
            ssem.at[slot])

    def wait_gather(chunk, slot):
        pltpu.make_async_copy(
            table_hbm.at[idx_v.at[chunk]], rows_v.at[slot, 0], gsem.at[slot]).wait()

    def wait_store(chunk, slot):
        pltpu.make_async_copy(
            rows_v.at[slot], out_hbm.at[pl.ds(blk0 + chunk, 1)],
            ssem.at[slot]).wait()

    for c in range(LA):
        fire_gather(c, c)
    for j in range(2):
        wait_gather(j, j)
        fire_store(j, j)
        fire_gather(j + LA, (j + LA) % NBUF)

    def group(g, carry):
        for k in range(NBUF):
            j = 2 + g * NBUF + k
            b = (2 + k) % NBUF
            s2 = (b + LA) % NBUF
            wait_gather(j, b)
            fire_store(j, b)
            wait_store(j - 2, s2)
            fire_gather(j + LA, s2)
        return carry

    G = (NCHUNK - 2 - LA) // NBUF
    lax.fori_loop(0, G, group, 0)

    for j in range(2 + G * NBUF, NCHUNK - LA):
        b = j % NBUF
        s2 = (b + LA) % NBUF
        wait_gather(j, b)
        fire_store(j, b)
        wait_store(j - 2, s2)
        fire_gather(j + LA, s2)
    for j in range(NCHUNK - LA, NCHUNK):
        b = j % NBUF
        wait_gather(j, b)
        fire_store(j, b)
        wait_store(j - 2, (b + LA) % NBUF)
    for j in range(NCHUNK - 2, NCHUNK):
        wait_store(j, j % NBUF)


@functools.partial(jax.jit, static_argnames=())
def _sc_gather(idx, table):
    kern = pl.kernel(
        _gather_body,
        out_type=jax.ShapeDtypeStruct((NBLK, CHUNK, D), jnp.float32),
        mesh=plsc.VectorSubcoreMesh(
            core_axis_name="c", subcore_axis_name="s",
            num_cores=NC, num_subcores=NS),
        scratch_types=[
            pltpu.VMEM((NCHUNK, CHUNK), jnp.int32),
            pltpu.VMEM((NBUF, 1, CHUNK, D), jnp.float32),
            pltpu.SemaphoreType.DMA((NBUF,)),
            pltpu.SemaphoreType.DMA((NBUF,)),
        ],
    )
    return kern(idx, table)


def kernel(distance, FEATURE):
    idx = jnp.transpose(distance).reshape(NW, NCHUNK, CHUNK).astype(jnp.int32)
    out = _sc_gather(idx, FEATURE)
    return out.reshape(N_COLS, N_ROWS, D).transpose(1, 0, 2)
